# Initial kernel scaffold; baseline (speedup 1.0000x reference)
import jax
import jax.numpy as jnp
from jax.experimental import pallas as pl


def kernel(x, in1_w1, in1_b1, in1_w2, in1_b2, out1_w1, out1_b1, out1_w2, out1_b2, in2_w1, in2_b1, in2_w2, in2_b2, out2_w1, out2_b1, out2_w2, out2_b2, fc_w, fc_b):
    raise NotImplementedError("write your pallas kernel here")


# speedup vs baseline: 25.4361x; 25.4361x over previous
import jax
import jax.numpy as jnp
from jax.experimental import pallas as pl
from jax.experimental.pallas import tpu as pltpu

_K = 5
_NEG = -3.0e38


def _gelu(x):
    return 0.5 * x * (1.0 + jax.lax.erf(x * jnp.float32(0.7071067811865476)))


def _tln(x, w1, b1, w2, b2):
    return _gelu(x @ w1 + b1) @ w2 + b2


def _vig_kernel(x_ref,
                i1w1, i1b1, i1w2, i1b2,
                o1w1, o1b1, o1w2, o1b2,
                i2w1, i2b1, i2w2, i2b2,
                o2w1, o2b1, o2w2, o2b2,
                fch, fca, fcb,
                out_ref):
    x = x_ref[0]
    n = x.shape[0]

    h = _tln(x, i1w1[...], i1b1[...], i1w2[...], i1b2[...])

    sim = jax.lax.dot_general(x, x, (((1,), (1,)), ((), ())),
                              preferred_element_type=jnp.float32)
    iota = jax.lax.broadcasted_iota(jnp.int32, sim.shape, 1)

    acc = jnp.full(h.shape, _NEG, jnp.float32)
    for _ in range(_K):
        m = jnp.max(sim, axis=1, keepdims=True)
        idx = jnp.min(jnp.where(sim == m, iota, n), axis=1, keepdims=True)
        onehot = (iota == idx).astype(jnp.float32)
        g = jax.lax.dot_general(onehot, h, (((1,), (0,)), ((), ())),
                                preferred_element_type=jnp.float32)
        acc = jnp.maximum(acc, g)
        sim = jnp.where(iota == idx, _NEG, sim)

    agg = acc - h
    y = h @ fch[...] + agg @ fca[...] + fcb[...]
    y = jnp.maximum(y, 0.0)
    y = _tln(y, o1w1[...], o1b1[...], o1w2[...], o1b2[...])
    hh = y + x
    t = _tln(hh, i2w1[...], i2b1[...], i2w2[...], i2b2[...])
    t = jnp.maximum(t, 0.0)
    t = _tln(t, o2w1[...], o2b1[...], o2w2[...], o2b2[...])
    out_ref[0] = t + hh


def kernel(x, in1_w1, in1_b1, in1_w2, in1_b2, out1_w1, out1_b1, out1_w2, out1_b2,
           in2_w1, in2_b1, in2_w2, in2_b2, out2_w1, out2_b1, out2_w2, out2_b2,
           fc_w, fc_b):
    Bn, Nn, Cn = x.shape
    fch = fc_w[0::2]
    fca = fc_w[1::2]

    w_spec = pl.BlockSpec((Cn, Cn), lambda b: (0, 0))
    b_spec = pl.BlockSpec((1, Cn), lambda b: (0, 0))
    biases2d = lambda v: v.reshape(1, Cn)

    out = pl.pallas_call(
        _vig_kernel,
        grid=(Bn,),
        in_specs=[
            pl.BlockSpec((1, Nn, Cn), lambda b: (b, 0, 0)),
            w_spec, b_spec, w_spec, b_spec,
            w_spec, b_spec, w_spec, b_spec,
            w_spec, b_spec, w_spec, b_spec,
            w_spec, b_spec, w_spec, b_spec,
            w_spec, w_spec, b_spec,
        ],
        out_specs=pl.BlockSpec((1, Nn, Cn), lambda b: (b, 0, 0)),
        out_shape=jax.ShapeDtypeStruct((Bn, Nn, Cn), jnp.float32),
        compiler_params=pltpu.CompilerParams(
            dimension_semantics=("parallel",),
        ),
    )(x,
      in1_w1, biases2d(in1_b1), in1_w2, biases2d(in1_b2),
      out1_w1, biases2d(out1_b1), out1_w2, biases2d(out1_b2),
      in2_w1, biases2d(in2_b1), in2_w2, biases2d(in2_b2),
      out2_w1, biases2d(out2_b1), out2_w2, biases2d(out2_b2),
      fch, fca, biases2d(fc_b))
    return out


# speedup vs baseline: 32.7244x; 1.2865x over previous
import jax
import jax.numpy as jnp
from jax.experimental import pallas as pl
from jax.experimental.pallas import tpu as pltpu

_K = 5
_NEG = -3.0e38


def _gelu(x):
    return 0.5 * x * (1.0 + jax.lax.erf(x * jnp.float32(0.7071067811865476)))


def _tln(x, w1, b1, w2, b2):
    return _gelu(x @ w1 + b1) @ w2 + b2


def _vig_kernel(x_ref,
                i1w1, i1b1, i1w2, i1b2,
                o1w1, o1b1, o1w2, o1b2,
                i2w1, i2b1, i2w2, i2b2,
                o2w1, o2b1, o2w2, o2b2,
                fch, fca, fcb,
                out_ref):
    x = x_ref[0]
    n = x.shape[0]

    h = _tln(x, i1w1[...], i1b1[...], i1w2[...], i1b2[...])

    sim = jax.lax.dot_general(x, x, (((1,), (1,)), ((), ())),
                              preferred_element_type=jnp.float32)
    acc = jnp.full(h.shape, _NEG, jnp.float32)
    for k in range(_K):
        m = jnp.max(sim, axis=1, keepdims=True)
        eq = sim == m
        g = jax.lax.dot_general(jnp.where(eq, 1.0, 0.0), h,
                                (((1,), (0,)), ((), ())),
                                preferred_element_type=jnp.float32)
        acc = jnp.maximum(acc, g)
        if k < _K - 1:
            sim = jnp.where(eq, _NEG, sim)

    agg = acc - h
    y = h @ fch[...] + agg @ fca[...] + fcb[...]
    y = jnp.maximum(y, 0.0)
    y = _tln(y, o1w1[...], o1b1[...], o1w2[...], o1b2[...])
    hh = y + x
    t = _tln(hh, i2w1[...], i2b1[...], i2w2[...], i2b2[...])
    t = jnp.maximum(t, 0.0)
    t = _tln(t, o2w1[...], o2b1[...], o2w2[...], o2b2[...])
    out_ref[0] = t + hh


def kernel(x, in1_w1, in1_b1, in1_w2, in1_b2, out1_w1, out1_b1, out1_w2, out1_b2,
           in2_w1, in2_b1, in2_w2, in2_b2, out2_w1, out2_b1, out2_w2, out2_b2,
           fc_w, fc_b):
    Bn, Nn, Cn = x.shape
    fch = fc_w[0::2]
    fca = fc_w[1::2]

    w_spec = pl.BlockSpec((Cn, Cn), lambda b: (0, 0))
    b_spec = pl.BlockSpec((1, Cn), lambda b: (0, 0))
    biases2d = lambda v: v.reshape(1, Cn)

    out = pl.pallas_call(
        _vig_kernel,
        grid=(Bn,),
        in_specs=[
            pl.BlockSpec((1, Nn, Cn), lambda b: (b, 0, 0)),
            w_spec, b_spec, w_spec, b_spec,
            w_spec, b_spec, w_spec, b_spec,
            w_spec, b_spec, w_spec, b_spec,
            w_spec, b_spec, w_spec, b_spec,
            w_spec, w_spec, b_spec,
        ],
        out_specs=pl.BlockSpec((1, Nn, Cn), lambda b: (b, 0, 0)),
        out_shape=jax.ShapeDtypeStruct((Bn, Nn, Cn), jnp.float32),
        compiler_params=pltpu.CompilerParams(
            dimension_semantics=("parallel",),
        ),
    )(x,
      in1_w1, biases2d(in1_b1), in1_w2, biases2d(in1_b2),
      out1_w1, biases2d(out1_b1), out1_w2, biases2d(out1_b2),
      in2_w1, biases2d(in2_b1), in2_w2, biases2d(in2_b2),
      out2_w1, biases2d(out2_b1), out2_w2, biases2d(out2_b2),
      fch, fca, biases2d(fc_b))
    return out


# speedup vs baseline: 33.2313x; 1.0155x over previous
import jax
import jax.numpy as jnp
from jax.experimental import pallas as pl
from jax.experimental.pallas import tpu as pltpu

_K = 5
_NEG = -3.0e38


def _gelu(x):
    return 0.5 * x * (1.0 + jax.lax.erf(x * jnp.float32(0.7071067811865476)))


def _tln(x, w1, b1, w2, b2):
    return _gelu(x @ w1 + b1) @ w2 + b2


def _vig_kernel(x_ref,
                i1w1, i1b1, i1w2, i1b2,
                o1w1, o1b1, o1w2, o1b2,
                i2w1, i2b1, i2w2, i2b2,
                o2w1, o2b1, o2w2, o2b2,
                fch, fca, fcb,
                out_ref):
    for bi in range(x_ref.shape[0]):
        x = x_ref[bi]

        h = _tln(x, i1w1[...], i1b1[...], i1w2[...], i1b2[...])

        sim = jax.lax.dot_general(x, x, (((1,), (1,)), ((), ())),
                                  preferred_element_type=jnp.float32)
        acc = jnp.full(h.shape, _NEG, jnp.float32)
        for k in range(_K):
            m = jnp.max(sim, axis=1, keepdims=True)
            eq = sim == m
            g = jax.lax.dot_general(jnp.where(eq, 1.0, 0.0), h,
                                    (((1,), (0,)), ((), ())),
                                    preferred_element_type=jnp.float32)
            acc = jnp.maximum(acc, g)
            if k < _K - 1:
                sim = jnp.where(eq, _NEG, sim)

        agg = acc - h
        y = h @ fch[...] + agg @ fca[...] + fcb[...]
        y = jnp.maximum(y, 0.0)
        y = _tln(y, o1w1[...], o1b1[...], o1w2[...], o1b2[...])
        hh = y + x
        t = _tln(hh, i2w1[...], i2b1[...], i2w2[...], i2b2[...])
        t = jnp.maximum(t, 0.0)
        t = _tln(t, o2w1[...], o2b1[...], o2w2[...], o2b2[...])
        out_ref[bi] = t + hh


def kernel(x, in1_w1, in1_b1, in1_w2, in1_b2, out1_w1, out1_b1, out1_w2, out1_b2,
           in2_w1, in2_b1, in2_w2, in2_b2, out2_w1, out2_b1, out2_w2, out2_b2,
           fc_w, fc_b):
    Bn, Nn, Cn = x.shape
    fch = fc_w[0::2]
    fca = fc_w[1::2]

    BPP = 2
    w_spec = pl.BlockSpec((Cn, Cn), lambda b: (0, 0))
    b_spec = pl.BlockSpec((1, Cn), lambda b: (0, 0))
    biases2d = lambda v: v.reshape(1, Cn)

    out = pl.pallas_call(
        _vig_kernel,
        grid=(Bn // BPP,),
        in_specs=[
            pl.BlockSpec((BPP, Nn, Cn), lambda b: (b, 0, 0)),
            w_spec, b_spec, w_spec, b_spec,
            w_spec, b_spec, w_spec, b_spec,
            w_spec, b_spec, w_spec, b_spec,
            w_spec, b_spec, w_spec, b_spec,
            w_spec, w_spec, b_spec,
        ],
        out_specs=pl.BlockSpec((BPP, Nn, Cn), lambda b: (b, 0, 0)),
        out_shape=jax.ShapeDtypeStruct((Bn, Nn, Cn), jnp.float32),
        compiler_params=pltpu.CompilerParams(
            dimension_semantics=("parallel",),
        ),
    )(x,
      in1_w1, biases2d(in1_b1), in1_w2, biases2d(in1_b2),
      out1_w1, biases2d(out1_b1), out1_w2, biases2d(out1_b2),
      in2_w1, biases2d(in2_b1), in2_w2, biases2d(in2_b2),
      out2_w1, biases2d(out2_b1), out2_w2, biases2d(out2_b2),
      fch, fca, biases2d(fc_b))
    return out


# speedup vs baseline: 36.2464x; 1.0907x over previous
import jax
import jax.numpy as jnp
from jax.experimental import pallas as pl
from jax.experimental.pallas import tpu as pltpu

_K = 5
_NEG = -3.0e38


def _gelu(x):
    return 0.5 * x * (1.0 + jax.lax.erf(x * jnp.float32(0.7071067811865476)))


def _tln(x, w1, b1, w2, b2):
    return _gelu(x @ w1 + b1) @ w2 + b2


def _vig_kernel(x_ref,
                i1w1, i1b1, i1w2, i1b2,
                o1w1, o1b1, o1w2, o1b2,
                i2w1, i2b1, i2w2, i2b2,
                o2w1, o2b1, o2w2, o2b2,
                fch, fca, fcb,
                out_ref):
    bpp, n, c = x_ref.shape
    x = x_ref[...].reshape(bpp * n, c)

    h = _tln(x, i1w1[...], i1b1[...], i1w2[...], i1b2[...])

    sim = jnp.concatenate([
        jax.lax.dot_general(x[bi * n:(bi + 1) * n], x[bi * n:(bi + 1) * n],
                            (((1,), (1,)), ((), ())),
                            preferred_element_type=jnp.float32)
        for bi in range(bpp)], axis=0)

    acc = jnp.full(h.shape, _NEG, jnp.float32)
    for k in range(_K):
        m = jnp.max(sim, axis=1, keepdims=True)
        eq = sim == m
        oh = jnp.where(eq, 1.0, 0.0)
        g = jnp.concatenate([
            jax.lax.dot_general(oh[bi * n:(bi + 1) * n], h[bi * n:(bi + 1) * n],
                                (((1,), (0,)), ((), ())),
                                preferred_element_type=jnp.float32)
            for bi in range(bpp)], axis=0)
        acc = jnp.maximum(acc, g)
        if k < _K - 1:
            sim = jnp.where(eq, _NEG, sim)

    agg = acc - h
    y = h @ fch[...] + agg @ fca[...] + fcb[...]
    y = jnp.maximum(y, 0.0)
    y = _tln(y, o1w1[...], o1b1[...], o1w2[...], o1b2[...])
    hh = y + x
    t = _tln(hh, i2w1[...], i2b1[...], i2w2[...], i2b2[...])
    t = jnp.maximum(t, 0.0)
    t = _tln(t, o2w1[...], o2b1[...], o2w2[...], o2b2[...])
    out_ref[...] = (t + hh).reshape(bpp, n, c)


def kernel(x, in1_w1, in1_b1, in1_w2, in1_b2, out1_w1, out1_b1, out1_w2, out1_b2,
           in2_w1, in2_b1, in2_w2, in2_b2, out2_w1, out2_b1, out2_w2, out2_b2,
           fc_w, fc_b):
    Bn, Nn, Cn = x.shape
    fch = fc_w[0::2]
    fca = fc_w[1::2]

    BPP = 2
    w_spec = pl.BlockSpec((Cn, Cn), lambda b: (0, 0))
    b_spec = pl.BlockSpec((1, Cn), lambda b: (0, 0))
    biases2d = lambda v: v.reshape(1, Cn)

    out = pl.pallas_call(
        _vig_kernel,
        grid=(Bn // BPP,),
        in_specs=[
            pl.BlockSpec((BPP, Nn, Cn), lambda b: (b, 0, 0)),
            w_spec, b_spec, w_spec, b_spec,
            w_spec, b_spec, w_spec, b_spec,
            w_spec, b_spec, w_spec, b_spec,
            w_spec, b_spec, w_spec, b_spec,
            w_spec, w_spec, b_spec,
        ],
        out_specs=pl.BlockSpec((BPP, Nn, Cn), lambda b: (b, 0, 0)),
        out_shape=jax.ShapeDtypeStruct((Bn, Nn, Cn), jnp.float32),
        compiler_params=pltpu.CompilerParams(
            dimension_semantics=("parallel",),
        ),
    )(x,
      in1_w1, biases2d(in1_b1), in1_w2, biases2d(in1_b2),
      out1_w1, biases2d(out1_b1), out1_w2, biases2d(out1_b2),
      in2_w1, biases2d(in2_b1), in2_w2, biases2d(in2_b2),
      out2_w1, biases2d(out2_b1), out2_w2, biases2d(out2_b2),
      fch, fca, biases2d(fc_b))
    return out
